# baseline (speedup 1.0000x reference)
import functools

import jax
import jax.numpy as jnp
import numpy as np
from jax import lax
from jax.experimental import pallas as pl
from jax.experimental.pallas import tpu as pltpu
from jax.experimental.pallas import tpu_sc as plsc

N = 10000
E = 320000
D_IN = 128
HID = 256
HALF = 128
G = 64

NCORE = 2
NSUB = 16
LANES = 16
NPAD = 10240
ROWS_PER_TILE = NPAD // NSUB
CHUNK = 80
NCHUNK = 256
EPT = NCHUNK * CHUNK
EPAD = NSUB * EPT

_mesh = plsc.VectorSubcoreMesh(
    core_axis_name="c", subcore_axis_name="s",
    num_cores=NCORE, num_subcores=NSUB)


def _deg_body(dst_hbm, ew_hbm, zeros_hbm, deg_hbm, didx_v, val_v, acc_sh, ssem):
    c = lax.axis_index("c")
    s = lax.axis_index("s")

    @pl.when(c == 0)
    def _():
        sl = pl.ds(s * ROWS_PER_TILE, ROWS_PER_TILE)
        pltpu.sync_copy(zeros_hbm.at[sl], acc_sh.at[sl])
        row0 = s * NCHUNK
        pltpu.sync_copy(dst_hbm.at[pl.ds(row0, NCHUNK)], didx_v)
        pltpu.sync_copy(ew_hbm.at[pl.ds(row0, NCHUNK)], val_v)
        plsc.subcore_barrier()

        def fire(i, carry):
            pltpu.async_copy(val_v.at[i], acc_sh.at[didx_v.at[i]], ssem,
                             add=True)
            return carry

        lax.fori_loop(0, NCHUNK, fire, 0)

        def drain(i, carry):
            pltpu.make_async_copy(val_v.at[0], acc_sh.at[didx_v.at[0]],
                                  ssem).wait()
            return carry

        lax.fori_loop(0, NCHUNK, drain, 0)
        plsc.subcore_barrier()
        pltpu.sync_copy(acc_sh.at[sl], deg_hbm.at[sl])


_deg_call = pl.kernel(
    _deg_body,
    out_type=jax.ShapeDtypeStruct((NPAD,), jnp.float32),
    mesh=_mesh,
    scratch_types=[
        pltpu.VMEM((NCHUNK, CHUNK), jnp.int32),
        pltpu.VMEM((NCHUNK, CHUNK), jnp.float32),
        pltpu.VMEM_SHARED((NPAD,), jnp.float32),
        pltpu.SemaphoreType.DMA,
    ],
)


IB = 16
NSUP = NCHUNK // IB
NBUF = 4


def _agg_body(table_hbm, src_hbm, dst_hbm, ew_hbm, zeros_hbm, out_hbm,
              sidx_v, didx_v, ew_v, r0, r1, r2, r3,
              acc_sh, g0, g1, g2, g3, s0, s1, s2, s3):
    c = lax.axis_index("c")
    s = lax.axis_index("s")
    sl = pl.ds(s * ROWS_PER_TILE, ROWS_PER_TILE)
    pltpu.sync_copy(zeros_hbm.at[sl], acc_sh.at[sl])
    plsc.subcore_barrier()

    rows = (r0, r1, r2, r3)
    gs = (g0, g1, g2, g3)
    ss = (s0, s1, s2, s3)
    roff = lax.broadcast(c * NPAD, (LANES,))

    def sup(k, carry):
        row0 = s * NCHUNK + k * IB
        pltpu.sync_copy(src_hbm.at[pl.ds(row0, IB)], sidx_v)
        pltpu.sync_copy(dst_hbm.at[pl.ds(row0, IB)], didx_v)
        pltpu.sync_copy(ew_hbm.at[pl.ds(row0, IB)], ew_v)

        @pl.when(c == 1)
        def _():
            def adj(i, carry2):
                for j in range(CHUNK // LANES):
                    jsl = pl.ds(j * LANES, LANES)
                    sidx_v[i, jsl] = sidx_v[i, jsl] + roff
                return carry2

            lax.fori_loop(0, IB, adj, 0)

        for j in range(NBUF - 1):
            pltpu.async_copy(table_hbm.at[sidx_v.at[j]], rows[j], gs[j])

        def outer(g, carry2):
            for b in range(NBUF):
                i = NBUF * g + b
                pltpu.make_async_copy(table_hbm.at[sidx_v.at[i]], rows[b],
                                      gs[b]).wait()

                def sgrp(gr, carry3):
                    w16 = ew_v[i, pl.ds(gr * LANES, LANES)]
                    for l in range(LANES):
                        e = gr * LANES + l
                        w = lax.broadcast(w16[l], (LANES,))
                        for j in range(HALF // LANES):
                            fsl = pl.ds(j * LANES, LANES)
                            rows[b][e, fsl] = rows[b][e, fsl] * w
                    return carry3

                lax.fori_loop(0, CHUNK // LANES, sgrp, 0)
                pltpu.async_copy(rows[b], acc_sh.at[didx_v.at[i]], ss[b],
                                 add=True)

                pb = (b - 1) % NBUF

                @pl.when(i >= 1)
                def _():
                    pltpu.make_async_copy(rows[pb],
                                          acc_sh.at[didx_v.at[i - 1]],
                                          ss[pb]).wait()

                @pl.when(i + NBUF - 1 < IB)
                def _():
                    pltpu.async_copy(table_hbm.at[sidx_v.at[i + NBUF - 1]],
                                     rows[pb], gs[pb])
            return carry2

        lax.fori_loop(0, IB // NBUF, outer, 0)
        pltpu.make_async_copy(rows[(IB - 1) % NBUF],
                              acc_sh.at[didx_v.at[IB - 1]],
                              ss[(IB - 1) % NBUF]).wait()
        return carry

    lax.fori_loop(0, NSUP, sup, 0)
    plsc.subcore_barrier()
    pltpu.sync_copy(acc_sh.at[sl], out_hbm.at[c, sl])


_agg_call = pl.kernel(
    _agg_body,
    out_type=jax.ShapeDtypeStruct((NCORE, NPAD, HALF), jnp.float32),
    mesh=_mesh,
    scratch_types=[
        pltpu.VMEM((IB, CHUNK), jnp.int32),
        pltpu.VMEM((IB, CHUNK), jnp.int32),
        pltpu.VMEM((IB, CHUNK), jnp.float32),
        pltpu.VMEM((CHUNK, HALF), jnp.float32),
        pltpu.VMEM((CHUNK, HALF), jnp.float32),
        pltpu.VMEM((CHUNK, HALF), jnp.float32),
        pltpu.VMEM((CHUNK, HALF), jnp.float32),
        pltpu.VMEM_SHARED((NPAD, HALF), jnp.float32),
        pltpu.SemaphoreType.DMA,
        pltpu.SemaphoreType.DMA,
        pltpu.SemaphoreType.DMA,
        pltpu.SemaphoreType.DMA,
        pltpu.SemaphoreType.DMA,
        pltpu.SemaphoreType.DMA,
        pltpu.SemaphoreType.DMA,
        pltpu.SemaphoreType.DMA,
    ],
)


RB = 1024
NBLK = NPAD // RB


def _dinv_of(deg):
    return jnp.where(deg > 0, lax.rsqrt(jnp.where(deg > 0, deg, 1.0)), 0.0)


def _c1_body(x_ref, deg_ref, w_ref, out_ref):
    dinv = _dinv_of(deg_ref[:, 0])
    hw = jnp.dot(x_ref[...], w_ref[...], preferred_element_type=jnp.float32)
    hws = hw * dinv[:, None]
    out_ref[0] = hws[:, :HALF]
    out_ref[1] = hws[:, HALF:]


_c1_call = pl.pallas_call(
    _c1_body,
    grid=(NBLK,),
    in_specs=[
        pl.BlockSpec((RB, D_IN), lambda i: (i, 0)),
        pl.BlockSpec((RB, 1), lambda i: (i, 0)),
        pl.BlockSpec((D_IN, HID), lambda i: (0, 0)),
    ],
    out_specs=pl.BlockSpec((NCORE, RB, HALF), lambda i: (0, i, 0)),
    out_shape=jax.ShapeDtypeStruct((NCORE, NPAD, HALF), jnp.float32),
)


def _silu(z):
    return z / (1.0 + jnp.exp(-z))


def _c2_body(agg_ref, deg_ref, b_ref, w_ref, out_ref):
    dinv = _dinv_of(deg_ref[:, 0])
    a = jnp.concatenate([agg_ref[0], agg_ref[1]], axis=1)
    z = a * dinv[:, None] + b_ref[0][None, :]
    h = _silu(z)
    hws = (jnp.dot(h, w_ref[...], preferred_element_type=jnp.float32)
           * dinv[:, None])
    out_ref[0] = hws[:, :HALF]
    out_ref[1] = hws[:, HALF:]


_c2_call = pl.pallas_call(
    _c2_body,
    grid=(NBLK,),
    in_specs=[
        pl.BlockSpec((NCORE, RB, HALF), lambda i: (0, i, 0)),
        pl.BlockSpec((RB, 1), lambda i: (i, 0)),
        pl.BlockSpec((1, HID), lambda i: (0, 0)),
        pl.BlockSpec((HID, HID), lambda i: (0, 0)),
    ],
    out_specs=pl.BlockSpec((NCORE, RB, HALF), lambda i: (0, i, 0)),
    out_shape=jax.ShapeDtypeStruct((NCORE, NPAD, HALF), jnp.float32),
)


def _e_body(agg_ref, deg_ref, b_ref, batch_ref, wp_ref, bp_ref, wf_ref, bf_ref,
            out_ref, pool_acc, cnt_acc):
    i = pl.program_id(0)

    @pl.when(i == 0)
    def _():
        pool_acc[...] = jnp.zeros_like(pool_acc)
        cnt_acc[...] = jnp.zeros_like(cnt_acc)

    dinv = _dinv_of(deg_ref[:, 0])
    a = jnp.concatenate([agg_ref[0], agg_ref[1]], axis=1)
    z = a * dinv[:, None] + b_ref[0][None, :]
    h = _silu(z)
    bt = batch_ref[:, 0]
    onehot = (bt[:, None] == lax.broadcasted_iota(jnp.int32, (RB, G), 1)
              ).astype(jnp.float32)
    pool_acc[...] += lax.dot_general(onehot, h, (((0,), (0,)), ((), ())),
                                     preferred_element_type=jnp.float32)
    cnt_acc[...] += jnp.sum(onehot, axis=0, keepdims=True)

    @pl.when(i == pl.num_programs(0) - 1)
    def _():
        pooled = pool_acc[...] / jnp.maximum(cnt_acc[0], 1.0)[:, None]
        hp = jnp.dot(pooled, wp_ref[...], preferred_element_type=jnp.float32)
        hp = hp + bp_ref[0][None, :]
        hp = _silu(hp)
        out_ref[...] = jnp.dot(hp, wf_ref[...],
                               preferred_element_type=jnp.float32) + bf_ref[0][None, :]


_e_call = pl.pallas_call(
    _e_body,
    grid=(NBLK,),
    in_specs=[
        pl.BlockSpec((NCORE, RB, HALF), lambda i: (0, i, 0)),
        pl.BlockSpec((RB, 1), lambda i: (i, 0)),
        pl.BlockSpec((1, HID), lambda i: (0, 0)),
        pl.BlockSpec((RB, 1), lambda i: (i, 0)),
        pl.BlockSpec((HID, HALF), lambda i: (0, 0)),
        pl.BlockSpec((1, HALF), lambda i: (0, 0)),
        pl.BlockSpec((HALF, 1), lambda i: (0, 0)),
        pl.BlockSpec((1, 1), lambda i: (0, 0)),
    ],
    out_specs=pl.BlockSpec((G, 1), lambda i: (0, 0)),
    out_shape=jax.ShapeDtypeStruct((G, 1), jnp.float32),
    scratch_shapes=[
        pltpu.VMEM((G, HID), jnp.float32),
        pltpu.VMEM((1, G), jnp.float32),
    ],
)


def kernel(x, edge_index, edge_weight, batch, W1, b1, W2, b2, Wp, bp, Wf, bf):
    src = edge_index[0]
    dst = edge_index[1]
    x_p = jnp.zeros((NPAD, D_IN), jnp.float32).at[:N].set(x)
    src_p = jnp.zeros((EPAD,), jnp.int32).at[:E].set(src).reshape(-1, CHUNK)
    dst_p = jnp.zeros((EPAD,), jnp.int32).at[:E].set(dst).reshape(-1, CHUNK)
    ew_p = jnp.zeros((EPAD,), jnp.float32).at[:E].set(edge_weight).reshape(-1, CHUNK)
    batch_p = jnp.full((NPAD,), G, jnp.int32).at[:N].set(batch).reshape(NPAD, 1)
    zeros_n = jnp.zeros((NPAD,), jnp.float32)
    zeros_rows = jnp.zeros((NPAD, HALF), jnp.float32)

    deg = _deg_call(dst_p, ew_p, zeros_n)
    deg2 = deg.reshape(NPAD, 1)

    tbl = lambda t: t.reshape(NCORE * NPAD, HALF)
    t = _c1_call(x_p, deg2, W1)
    agg = _agg_call(tbl(t), src_p, dst_p, ew_p, zeros_rows)
    t = _c2_call(agg, deg2, b1.reshape(1, HID), W2)
    agg = _agg_call(tbl(t), src_p, dst_p, ew_p, zeros_rows)
    t = _c2_call(agg, deg2, b2.reshape(1, HID), W2)
    agg = _agg_call(tbl(t), src_p, dst_p, ew_p, zeros_rows)

    out = _e_call(agg, deg2, b2.reshape(1, HID), batch_p,
                  Wp, bp.reshape(1, HALF), Wf, bf.reshape(1, 1))
    return out


# speedup vs baseline: 1.1493x; 1.1493x over previous
import functools

import jax
import jax.numpy as jnp
import numpy as np
from jax import lax
from jax.experimental import pallas as pl
from jax.experimental.pallas import tpu as pltpu
from jax.experimental.pallas import tpu_sc as plsc

N = 10000
E = 320000
D_IN = 128
HID = 256
HALF = 128
G = 64

NCORE = 2
NSUB = 16
LANES = 16
NPAD = 10240
ROWS_PER_TILE = NPAD // NSUB
CHUNK = 128
NCHUNK = 160
EPT = NCHUNK * CHUNK
EPAD = NSUB * EPT

_mesh = plsc.VectorSubcoreMesh(
    core_axis_name="c", subcore_axis_name="s",
    num_cores=NCORE, num_subcores=NSUB)


def _deg_body(dst_hbm, ew_hbm, zeros_hbm, deg_hbm, didx_v, val_v, acc_sh, ssem):
    c = lax.axis_index("c")
    s = lax.axis_index("s")

    @pl.when(c == 0)
    def _():
        sl = pl.ds(s * ROWS_PER_TILE, ROWS_PER_TILE)
        pltpu.sync_copy(zeros_hbm.at[sl], acc_sh.at[sl])
        row0 = s * NCHUNK
        pltpu.sync_copy(dst_hbm.at[pl.ds(row0, NCHUNK)], didx_v)
        pltpu.sync_copy(ew_hbm.at[pl.ds(row0, NCHUNK)], val_v)
        plsc.subcore_barrier()

        def fire(i, carry):
            pltpu.async_copy(val_v.at[i], acc_sh.at[didx_v.at[i]], ssem,
                             add=True)
            return carry

        lax.fori_loop(0, NCHUNK, fire, 0)

        def drain(i, carry):
            pltpu.make_async_copy(val_v.at[0], acc_sh.at[didx_v.at[0]],
                                  ssem).wait()
            return carry

        lax.fori_loop(0, NCHUNK, drain, 0)
        plsc.subcore_barrier()
        pltpu.sync_copy(acc_sh.at[sl], deg_hbm.at[sl])


_deg_call = pl.kernel(
    _deg_body,
    out_type=jax.ShapeDtypeStruct((NPAD,), jnp.float32),
    mesh=_mesh,
    scratch_types=[
        pltpu.VMEM((NCHUNK, CHUNK), jnp.int32),
        pltpu.VMEM((NCHUNK, CHUNK), jnp.float32),
        pltpu.VMEM_SHARED((NPAD,), jnp.float32),
        pltpu.SemaphoreType.DMA,
    ],
)


IB = 8
NSUP = NCHUNK // IB


def _agg_body(table_hbm, src_hbm, dst_hbm, ew_hbm, zeros_hbm, out_hbm,
              sA_s, sA_d, sA_w, sB_s, sB_d, sB_w, r0, r1,
              acc_sh, g0, g1, s0, s1, st0, st1):
    c = lax.axis_index("c")
    s = lax.axis_index("s")
    sl = pl.ds(s * ROWS_PER_TILE, ROWS_PER_TILE)
    pltpu.sync_copy(zeros_hbm.at[sl], acc_sh.at[sl])
    plsc.subcore_barrier()

    rows = (r0, r1)
    gs = (g0, g1)
    ss = (s0, s1)
    stg = ((sA_s, sA_d, sA_w), (sB_s, sB_d, sB_w))
    stsem = (st0, st1)
    roff = lax.broadcast(c * NPAD, (LANES,))

    def issue_staging(k, se):
        row0 = s * NCHUNK + k * IB
        pltpu.async_copy(src_hbm.at[pl.ds(row0, IB)], stg[se][0], stsem[se])
        pltpu.async_copy(dst_hbm.at[pl.ds(row0, IB)], stg[se][1], stsem[se])
        pltpu.async_copy(ew_hbm.at[pl.ds(row0, IB)], stg[se][2], stsem[se])

    def wait_staging(se):
        for src_ref, dst_ref in zip((src_hbm, dst_hbm, ew_hbm), stg[se]):
            pltpu.make_async_copy(src_ref.at[pl.ds(0, IB)], dst_ref,
                                  stsem[se]).wait()

    def adjust(se):
        @pl.when(c == 1)
        def _():
            def adj(i, carry):
                for j in range(CHUNK // LANES):
                    jsl = pl.ds(j * LANES, LANES)
                    stg[se][0][i, jsl] = stg[se][0][i, jsl] + roff
                return carry

            lax.fori_loop(0, IB, adj, 0)

    def wait_scatter(b, se):
        pltpu.make_async_copy(rows[b], acc_sh.at[stg[se][1].at[0]],
                              ss[b]).wait()

    def wait_gather(b, se):
        pltpu.make_async_copy(table_hbm.at[stg[se][0].at[0]], rows[b],
                              gs[b]).wait()

    issue_staging(0, 0)
    wait_staging(0)
    adjust(0)
    pltpu.async_copy(table_hbm.at[stg[0][0].at[0]], rows[0], gs[0])

    def supouter(k2, carry):
        for ks in range(2):
            k = 2 * k2 + ks
            se = ks
            oe = 1 - ks
            sidx, didx, ewst = stg[se]

            def inner(g, carry2):
                for b in range(2):
                    i = 2 * g + b
                    nb = 1 - b

                    if ks == 0:
                        @pl.when((k2 > 0) | (g > 0) | (b > 0))
                        def _():
                            wait_scatter(nb, se)
                    else:
                        wait_scatter(nb, se)

                    if b == 0:
                        @pl.when((g == 0) & (k + 1 < NSUP))
                        def _():
                            issue_staging(k + 1, oe)

                        @pl.when((g == 2) & (k + 1 < NSUP))
                        def _():
                            wait_staging(oe)
                            adjust(oe)

                    @pl.when(i + 1 < IB)
                    def _():
                        pltpu.async_copy(table_hbm.at[sidx.at[i + 1]],
                                         rows[nb], gs[nb])
                    if b == 1:
                        @pl.when((g == IB // 2 - 1) & (k + 1 < NSUP))
                        def _():
                            pltpu.async_copy(table_hbm.at[stg[oe][0].at[0]],
                                             rows[0], gs[0])

                    wait_gather(b, se)

                    def sgrp(gr, carry3):
                        w16 = ewst[i, pl.ds(gr * LANES, LANES)]
                        for l in range(LANES):
                            e = gr * LANES + l
                            w = lax.broadcast(w16[l], (LANES,))
                            for j in range(HALF // LANES):
                                fsl = pl.ds(j * LANES, LANES)
                                rows[b][e, fsl] = rows[b][e, fsl] * w
                        return carry3

                    lax.fori_loop(0, CHUNK // LANES, sgrp, 0)
                    pltpu.async_copy(rows[b], acc_sh.at[didx.at[i]], ss[b],
                                     add=True)
                return carry2

            lax.fori_loop(0, IB // 2, inner, 0)
        return carry

    lax.fori_loop(0, NSUP // 2, supouter, 0)
    wait_scatter((NCHUNK - 1) % 2, (NSUP - 1) % 2)
    plsc.subcore_barrier()
    pltpu.sync_copy(acc_sh.at[sl], out_hbm.at[c, sl])


_agg_call = pl.kernel(
    _agg_body,
    out_type=jax.ShapeDtypeStruct((NCORE, NPAD, HALF), jnp.float32),
    mesh=_mesh,
    scratch_types=[
        pltpu.VMEM((IB, CHUNK), jnp.int32),
        pltpu.VMEM((IB, CHUNK), jnp.int32),
        pltpu.VMEM((IB, CHUNK), jnp.float32),
        pltpu.VMEM((IB, CHUNK), jnp.int32),
        pltpu.VMEM((IB, CHUNK), jnp.int32),
        pltpu.VMEM((IB, CHUNK), jnp.float32),
        pltpu.VMEM((CHUNK, HALF), jnp.float32),
        pltpu.VMEM((CHUNK, HALF), jnp.float32),
        pltpu.VMEM_SHARED((NPAD, HALF), jnp.float32),
        pltpu.SemaphoreType.DMA,
        pltpu.SemaphoreType.DMA,
        pltpu.SemaphoreType.DMA,
        pltpu.SemaphoreType.DMA,
        pltpu.SemaphoreType.DMA,
        pltpu.SemaphoreType.DMA,
    ],
)


RB = 1024
NBLK = NPAD // RB


def _dinv_of(deg):
    return jnp.where(deg > 0, lax.rsqrt(jnp.where(deg > 0, deg, 1.0)), 0.0)


def _c1_body(x_ref, deg_ref, w_ref, out_ref):
    dinv = _dinv_of(deg_ref[:, 0])
    hw = jnp.dot(x_ref[...], w_ref[...], preferred_element_type=jnp.float32)
    hws = hw * dinv[:, None]
    out_ref[0] = hws[:, :HALF]
    out_ref[1] = hws[:, HALF:]


_c1_call = pl.pallas_call(
    _c1_body,
    grid=(NBLK,),
    in_specs=[
        pl.BlockSpec((RB, D_IN), lambda i: (i, 0)),
        pl.BlockSpec((RB, 1), lambda i: (i, 0)),
        pl.BlockSpec((D_IN, HID), lambda i: (0, 0)),
    ],
    out_specs=pl.BlockSpec((NCORE, RB, HALF), lambda i: (0, i, 0)),
    out_shape=jax.ShapeDtypeStruct((NCORE, NPAD, HALF), jnp.float32),
)


def _silu(z):
    return z / (1.0 + jnp.exp(-z))


def _c2_body(agg_ref, deg_ref, b_ref, w_ref, out_ref):
    dinv = _dinv_of(deg_ref[:, 0])
    a = jnp.concatenate([agg_ref[0], agg_ref[1]], axis=1)
    z = a * dinv[:, None] + b_ref[0][None, :]
    h = _silu(z)
    hws = (jnp.dot(h, w_ref[...], preferred_element_type=jnp.float32)
           * dinv[:, None])
    out_ref[0] = hws[:, :HALF]
    out_ref[1] = hws[:, HALF:]


_c2_call = pl.pallas_call(
    _c2_body,
    grid=(NBLK,),
    in_specs=[
        pl.BlockSpec((NCORE, RB, HALF), lambda i: (0, i, 0)),
        pl.BlockSpec((RB, 1), lambda i: (i, 0)),
        pl.BlockSpec((1, HID), lambda i: (0, 0)),
        pl.BlockSpec((HID, HID), lambda i: (0, 0)),
    ],
    out_specs=pl.BlockSpec((NCORE, RB, HALF), lambda i: (0, i, 0)),
    out_shape=jax.ShapeDtypeStruct((NCORE, NPAD, HALF), jnp.float32),
)


def _e_body(agg_ref, deg_ref, b_ref, batch_ref, wp_ref, bp_ref, wf_ref, bf_ref,
            out_ref, pool_acc, cnt_acc):
    i = pl.program_id(0)

    @pl.when(i == 0)
    def _():
        pool_acc[...] = jnp.zeros_like(pool_acc)
        cnt_acc[...] = jnp.zeros_like(cnt_acc)

    dinv = _dinv_of(deg_ref[:, 0])
    a = jnp.concatenate([agg_ref[0], agg_ref[1]], axis=1)
    z = a * dinv[:, None] + b_ref[0][None, :]
    h = _silu(z)
    bt = batch_ref[:, 0]
    onehot = (bt[:, None] == lax.broadcasted_iota(jnp.int32, (RB, G), 1)
              ).astype(jnp.float32)
    pool_acc[...] += lax.dot_general(onehot, h, (((0,), (0,)), ((), ())),
                                     preferred_element_type=jnp.float32)
    cnt_acc[...] += jnp.sum(onehot, axis=0, keepdims=True)

    @pl.when(i == pl.num_programs(0) - 1)
    def _():
        pooled = pool_acc[...] / jnp.maximum(cnt_acc[0], 1.0)[:, None]
        hp = jnp.dot(pooled, wp_ref[...], preferred_element_type=jnp.float32)
        hp = hp + bp_ref[0][None, :]
        hp = _silu(hp)
        out_ref[...] = jnp.dot(hp, wf_ref[...],
                               preferred_element_type=jnp.float32) + bf_ref[0][None, :]


_e_call = pl.pallas_call(
    _e_body,
    grid=(NBLK,),
    in_specs=[
        pl.BlockSpec((NCORE, RB, HALF), lambda i: (0, i, 0)),
        pl.BlockSpec((RB, 1), lambda i: (i, 0)),
        pl.BlockSpec((1, HID), lambda i: (0, 0)),
        pl.BlockSpec((RB, 1), lambda i: (i, 0)),
        pl.BlockSpec((HID, HALF), lambda i: (0, 0)),
        pl.BlockSpec((1, HALF), lambda i: (0, 0)),
        pl.BlockSpec((HALF, 1), lambda i: (0, 0)),
        pl.BlockSpec((1, 1), lambda i: (0, 0)),
    ],
    out_specs=pl.BlockSpec((G, 1), lambda i: (0, 0)),
    out_shape=jax.ShapeDtypeStruct((G, 1), jnp.float32),
    scratch_shapes=[
        pltpu.VMEM((G, HID), jnp.float32),
        pltpu.VMEM((1, G), jnp.float32),
    ],
)


def kernel(x, edge_index, edge_weight, batch, W1, b1, W2, b2, Wp, bp, Wf, bf):
    src = edge_index[0]
    dst = edge_index[1]
    x_p = jnp.zeros((NPAD, D_IN), jnp.float32).at[:N].set(x)
    src_p = jnp.zeros((EPAD,), jnp.int32).at[:E].set(src).reshape(-1, CHUNK)
    dst_p = jnp.zeros((EPAD,), jnp.int32).at[:E].set(dst).reshape(-1, CHUNK)
    ew_p = jnp.zeros((EPAD,), jnp.float32).at[:E].set(edge_weight).reshape(-1, CHUNK)
    batch_p = jnp.full((NPAD,), G, jnp.int32).at[:N].set(batch).reshape(NPAD, 1)
    zeros_n = jnp.zeros((NPAD,), jnp.float32)
    zeros_rows = jnp.zeros((NPAD, HALF), jnp.float32)

    deg = _deg_call(dst_p, ew_p, zeros_n)
    deg2 = deg.reshape(NPAD, 1)

    tbl = lambda t: t.reshape(NCORE * NPAD, HALF)
    t = _c1_call(x_p, deg2, W1)
    agg = _agg_call(tbl(t), src_p, dst_p, ew_p, zeros_rows)
    t = _c2_call(agg, deg2, b1.reshape(1, HID), W2)
    agg = _agg_call(tbl(t), src_p, dst_p, ew_p, zeros_rows)
    t = _c2_call(agg, deg2, b2.reshape(1, HID), W2)
    agg = _agg_call(tbl(t), src_p, dst_p, ew_p, zeros_rows)

    out = _e_call(agg, deg2, b2.reshape(1, HID), batch_p,
                  Wp, bp.reshape(1, HALF), Wf, bf.reshape(1, 1))
    return out
